# baseline (speedup 1.0000x reference)
import functools

import jax
import jax.numpy as jnp
from jax import lax
from jax.experimental import pallas as pl
from jax.experimental.pallas import tpu as pltpu
from jax.experimental.pallas import tpu_sc as plsc

EMBED_DIM = 1024
NUM_HEADS = 16
NUM_POINTS = 4
HEAD_DIM = EMBED_DIM // NUM_HEADS
LQ = 2048
LV = 2048
BZ = 4

NC, NS, L = 2, 16, 16
NW = NC * NS

R_TOTAL = LQ * BZ * NUM_HEADS
ROWB = LQ * BZ
ENTRIES = ROWB * 128
RB_PER_CHUNK = 4
CHUNK_E = RB_PER_CHUNK * 128
CHUNK_R = RB_PER_CHUNK * NUM_HEADS
RB_PER_W = ROWB // NW
CHUNKS_PER_W = RB_PER_W // RB_PER_CHUNK


def _matmul_bias_kernel(x_ref, w_ref, b_ref, o_ref):
    o_ref[...] = (
        jnp.dot(x_ref[...], w_ref[...], preferred_element_type=jnp.float32)
        + b_ref[...]
    )


def _matmul_bias_mask_kernel(x_ref, w_ref, b_ref, m_ref, o_ref):
    o_ref[...] = (
        jnp.dot(x_ref[...], w_ref[...], preferred_element_type=jnp.float32)
        + b_ref[...]
    ) * m_ref[...]


def _matmul_bias(x, w_t, b, mask_col=None, bm=512):
    M, K = x.shape
    N = w_t.shape[1]
    grid = (M // bm,)
    if mask_col is None:
        return pl.pallas_call(
            _matmul_bias_kernel,
            grid=grid,
            in_specs=[
                pl.BlockSpec((bm, K), lambda i: (i, 0)),
                pl.BlockSpec((K, N), lambda i: (0, 0)),
                pl.BlockSpec((1, N), lambda i: (0, 0)),
            ],
            out_specs=pl.BlockSpec((bm, N), lambda i: (i, 0)),
            out_shape=jax.ShapeDtypeStruct((M, N), jnp.float32),
        )(x, w_t, b.reshape(1, N))
    return pl.pallas_call(
        _matmul_bias_mask_kernel,
        grid=grid,
        in_specs=[
            pl.BlockSpec((bm, K), lambda i: (i, 0)),
            pl.BlockSpec((K, N), lambda i: (0, 0)),
            pl.BlockSpec((1, N), lambda i: (0, 0)),
            pl.BlockSpec((bm, 1), lambda i: (i, 0)),
        ],
        out_specs=pl.BlockSpec((bm, N), lambda i: (i, 0)),
        out_shape=jax.ShapeDtypeStruct((M, N), jnp.float32),
    )(x, w_t, b.reshape(1, N), mask_col)


def _prep_kernel(bm, q_ref, w_ref, b_ref, rp_ref, idx_ref, coeff_ref):
    soaw = (
        jnp.dot(q_ref[...], w_ref[...], preferred_element_type=jnp.float32)
        + b_ref[...]
    )
    rp = rp_ref[...]
    ref_c = rp[:, 0:1]
    ref_w = rp[:, 1:2]
    i = pl.program_id(0)
    row = lax.broadcasted_iota(jnp.int32, (bm, 1), 0) + i * bm
    boff = (row % BZ) * NUM_HEADS
    h_iota = lax.broadcasted_iota(jnp.int32, (bm, NUM_HEADS), 1)

    a = [soaw[:, 64 + p * 16:64 + (p + 1) * 16] for p in range(4)]
    mx = jnp.maximum(jnp.maximum(a[0], a[1]), jnp.maximum(a[2], a[3]))
    e = [jnp.exp(x - mx) for x in a]
    inv = 1.0 / (e[0] + e[1] + e[2] + e[3])

    for p in range(4):
        x = (ref_c + soaw[:, p * 16:(p + 1) * 16] * (ref_w * 0.125)) * float(LV - 1)
        x0f = jnp.floor(x)
        w1 = x - x0f
        w0 = 1.0 - w1
        x0 = x0f.astype(jnp.int32)
        x1 = x0 + 1
        m0 = (x0 >= 0) & (x0 <= LV - 1)
        m1 = (x1 >= 0) & (x1 <= LV - 1)
        awp = e[p] * inv
        swap = x0 == -1
        c_lo = jnp.where(swap, awp * w1, jnp.where(m0, awp * w0, 0.0))
        c_hi = jnp.where(swap, 0.0, jnp.where(m1, awp * w1, 0.0))
        lv0 = jnp.clip(x0, 0, LV - 1)
        idx_ref[:, p * 16:(p + 1) * 16] = lv0 * (BZ * NUM_HEADS) + boff + h_iota
        coeff_ref[:, (2 * p) * 16:(2 * p + 1) * 16] = c_lo
        coeff_ref[:, (2 * p + 1) * 16:(2 * p + 2) * 16] = c_hi


def _prep(qf, w_cat, b_cat, refq, bm=512):
    M = qf.shape[0]
    grid = (M // bm,)
    return pl.pallas_call(
        functools.partial(_prep_kernel, bm),
        grid=grid,
        in_specs=[
            pl.BlockSpec((bm, EMBED_DIM), lambda i: (i, 0)),
            pl.BlockSpec((EMBED_DIM, 128), lambda i: (0, 0)),
            pl.BlockSpec((1, 128), lambda i: (0, 0)),
            pl.BlockSpec((bm, 2), lambda i: (i, 0)),
        ],
        out_specs=[
            pl.BlockSpec((bm, 64), lambda i: (i, 0)),
            pl.BlockSpec((bm, 128), lambda i: (i, 0)),
        ],
        out_shape=[
            jax.ShapeDtypeStruct((M, 64), jnp.int32),
            jax.ShapeDtypeStruct((M, 128), jnp.float32),
        ],
    )(qf, w_cat, b_cat.reshape(1, 128), refq)


def _sc_gather_kernel(table_hbm, idx_hbm, coeff_hbm, out_hbm,
                      idx_v, coeff_v, rows_v, out_v, sem):
    wid = lax.axis_index("s") * NC + lax.axis_index("c")

    def splat(vec, h_full):
        return lax.gather(
            vec, h_full[:, None],
            lax.GatherDimensionNumbers(
                offset_dims=(), collapsed_slice_dims=(0,),
                start_index_map=(0,)),
            slice_sizes=(1,),
            mode=lax.GatherScatterMode.PROMISE_IN_BOUNDS)

    def chunk_body(g, carry):
        rb0 = wid * RB_PER_W + g * RB_PER_CHUNK
        pltpu.sync_copy(idx_hbm.at[pl.ds(rb0, RB_PER_CHUNK), :], idx_v)
        pltpu.sync_copy(coeff_hbm.at[pl.ds(rb0 * 128, CHUNK_E)], coeff_v)
        descs = []
        for i in range(RB_PER_CHUNK):
            descs.append(pltpu.async_copy(
                table_hbm.at[idx_v.at[i]],
                rows_v.at[pl.ds(i * 64, 64), :],
                sem,
            ))
        for dsc in descs:
            dsc.wait()

        def row_body(r, carry2):
            rb = r // NUM_HEADS
            h = r % NUM_HEADS
            h_full = jnp.full((L,), h, jnp.int32)
            acc = [jnp.zeros((L,), jnp.float32) for _ in range(4)]
            for p in range(4):
                pos = rb * 64 + p * 16 + h
                c_lo = splat(coeff_v[pl.ds(rb * 128 + p * 32, L)], h_full)
                c_hi = splat(coeff_v[pl.ds(rb * 128 + p * 32 + 16, L)], h_full)
                for j in range(4):
                    acc[j] = (acc[j]
                              + c_lo * rows_v[pos, pl.ds(j * 16, L)]
                              + c_hi * rows_v[pos, pl.ds(64 + j * 16, L)])
            for j in range(4):
                out_v[r, pl.ds(j * 16, L)] = acc[j]
            return carry2

        lax.fori_loop(0, CHUNK_R, row_body, 0, unroll=2)
        pltpu.sync_copy(out_v, out_hbm.at[pl.ds(rb0 * NUM_HEADS, CHUNK_R), :])
        return carry

    lax.fori_loop(0, CHUNKS_PER_W, chunk_body, 0)


@functools.cache
def _sc_gather_fn():
    return pl.kernel(
        _sc_gather_kernel,
        out_type=jax.ShapeDtypeStruct((R_TOTAL, HEAD_DIM), jnp.float32),
        mesh=plsc.VectorSubcoreMesh(core_axis_name="c", subcore_axis_name="s",
                                    num_cores=NC, num_subcores=NS),
        scratch_types=[
            pltpu.VMEM((RB_PER_CHUNK, 64), jnp.int32),
            pltpu.VMEM((CHUNK_E,), jnp.float32),
            pltpu.VMEM((RB_PER_CHUNK * 64, 2 * HEAD_DIM), jnp.float32),
            pltpu.VMEM((CHUNK_R, HEAD_DIM), jnp.float32),
            pltpu.SemaphoreType.DMA,
        ],
    )


def _sc_gather(table, idx_all, coeff_all):
    return _sc_gather_fn()(table, idx_all, coeff_all.reshape(-1))


_PERM = [ (c % 16) * 4 + c // 16 for c in range(64) ]


def kernel(query, value, value_key_padding_mask, value_valid_ratio,
           reference_point, snippet_num, W_so, b_so, W_aw, b_aw, W_v, b_v,
           W_o, b_o):
    Lq, bz, d = query.shape
    Lv = value.shape[0]

    perm = jnp.array(_PERM, dtype=jnp.int32)
    w_cat = jnp.concatenate([W_so.T[:, perm], W_aw.T[:, perm]], axis=1)
    b_cat = jnp.concatenate([b_so[perm], b_aw[perm]])

    maskf = 1.0 - value_key_padding_mask.T.reshape(Lv * bz, 1).astype(jnp.float32)
    v3 = _matmul_bias(value.reshape(Lv * bz, d).astype(jnp.bfloat16),
                      W_v.T.astype(jnp.bfloat16), b_v,
                      mask_col=maskf).reshape(Lv, bz * NUM_HEADS, HEAD_DIM)
    nxt = jnp.concatenate(
        [v3[1:], jnp.zeros((1, bz * NUM_HEADS, HEAD_DIM), jnp.float32)], axis=0)
    table = jnp.concatenate([v3, nxt], axis=-1).reshape(R_TOTAL, 2 * HEAD_DIM)

    refq = jnp.transpose(reference_point, (1, 0, 2)).reshape(Lq * bz, 2)
    qf = query.reshape(Lq * bz, d)
    idx_all, coeff_all = _prep(qf, w_cat, b_cat, refq)

    attn = _sc_gather(table, idx_all, coeff_all)

    out = _matmul_bias(attn.reshape(Lq * bz, d).astype(jnp.bfloat16),
                       W_o.T.astype(jnp.bfloat16), b_o)
    return out.reshape(Lq, bz, d)


# speedup vs baseline: 1.3571x; 1.3571x over previous
import functools

import jax
import jax.numpy as jnp
from jax import lax
from jax.experimental import pallas as pl
from jax.experimental.pallas import tpu as pltpu
from jax.experimental.pallas import tpu_sc as plsc

EMBED_DIM = 1024
NUM_HEADS = 16
NUM_POINTS = 4
HEAD_DIM = EMBED_DIM // NUM_HEADS
LQ = 2048
LV = 2048
BZ = 4

NC, NS, L = 2, 16, 16
NW = NC * NS

R_TOTAL = LQ * BZ * NUM_HEADS
ROWB = LQ * BZ
ENTRIES = ROWB * 128
RB_PER_CHUNK = 4
CHUNK_E = RB_PER_CHUNK * 128
CHUNK_R = RB_PER_CHUNK * NUM_HEADS
RB_PER_W = ROWB // NW
CHUNKS_PER_W = RB_PER_W // RB_PER_CHUNK


def _mm(x, w):
    return lax.dot_general(x, w, (((1,), (1,)), ((), ())),
                           preferred_element_type=jnp.float32)


def _matmul_bias_kernel(x_ref, w_ref, b_ref, o_ref):
    o_ref[...] = _mm(x_ref[...], w_ref[...]) + b_ref[...]


def _matmul_bias_mask_kernel(x_ref, w_ref, b_ref, m_ref, o_ref):
    o_ref[...] = (_mm(x_ref[...], w_ref[...]) + b_ref[...]) * m_ref[...]


def _matmul_bias(x, w, b, mask_col=None, bm=512):
    M, K = x.shape
    N = w.shape[0]
    grid = (M // bm,)
    if mask_col is None:
        return pl.pallas_call(
            _matmul_bias_kernel,
            grid=grid,
            in_specs=[
                pl.BlockSpec((bm, K), lambda i: (i, 0)),
                pl.BlockSpec((N, K), lambda i: (0, 0)),
                pl.BlockSpec((1, N), lambda i: (0, 0)),
            ],
            out_specs=pl.BlockSpec((bm, N), lambda i: (i, 0)),
            out_shape=jax.ShapeDtypeStruct((M, N), jnp.float32),
        )(x, w, b.reshape(1, N))
    return pl.pallas_call(
        _matmul_bias_mask_kernel,
        grid=grid,
        in_specs=[
            pl.BlockSpec((bm, K), lambda i: (i, 0)),
            pl.BlockSpec((N, K), lambda i: (0, 0)),
            pl.BlockSpec((1, N), lambda i: (0, 0)),
            pl.BlockSpec((bm, 1), lambda i: (i, 0)),
        ],
        out_specs=pl.BlockSpec((bm, N), lambda i: (i, 0)),
        out_shape=jax.ShapeDtypeStruct((M, N), jnp.float32),
    )(x, w, b.reshape(1, N), mask_col)


def _pair_kernel(mm_ref, mmn_ref, o_ref):
    mm = mm_ref[...]
    for hh in range(2):
        lo = mm[:, hh * HEAD_DIM:(hh + 1) * HEAD_DIM]
        hi = jnp.concatenate(
            [mm[4:, hh * HEAD_DIM:(hh + 1) * HEAD_DIM],
             mmn_ref[0:4, hh * HEAD_DIM:(hh + 1) * HEAD_DIM]], axis=0)
        o_ref[hh, :, 0:HEAD_DIM] = lo
        o_ref[hh, :, HEAD_DIM:] = hi


def _pair_build(mm):
    nblk = LV // 128
    table3 = pl.pallas_call(
        _pair_kernel,
        grid=(NUM_HEADS // 2, nblk),
        in_specs=[
            pl.BlockSpec((512, 2 * HEAD_DIM), lambda hp, i: (i, hp)),
            pl.BlockSpec((8, 2 * HEAD_DIM),
                         lambda hp, i: (jnp.minimum(i + 1, nblk - 1) * 64, hp)),
        ],
        out_specs=pl.BlockSpec((2, 512, 2 * HEAD_DIM), lambda hp, i: (hp, i, 0)),
        out_shape=jax.ShapeDtypeStruct((NUM_HEADS, ROWB, 2 * HEAD_DIM),
                                       jnp.float32),
    )(mm, mm)
    return table3.reshape(R_TOTAL, 2 * HEAD_DIM)


def _prep_kernel(bm, q_ref, w_ref, b_ref, rp_ref, idx_ref, coeff_ref):
    soaw = _mm(q_ref[...], w_ref[...]) + b_ref[...]
    rp = rp_ref[...]
    ref_c = rp[:, 0:1]
    ref_w = rp[:, 1:2]
    i = pl.program_id(0)
    row = lax.broadcasted_iota(jnp.int32, (bm, 1), 0) + i * bm
    boff = row % BZ
    h_iota = lax.broadcasted_iota(jnp.int32, (bm, NUM_HEADS), 1)
    hoff = boff + h_iota * ROWB

    a = [soaw[:, 64 + p * 16:64 + (p + 1) * 16] for p in range(4)]
    mx = jnp.maximum(jnp.maximum(a[0], a[1]), jnp.maximum(a[2], a[3]))
    e = [jnp.exp(x - mx) for x in a]
    inv = 1.0 / (e[0] + e[1] + e[2] + e[3])

    for p in range(4):
        x = (ref_c + soaw[:, p * 16:(p + 1) * 16] * (ref_w * 0.125)) * float(LV - 1)
        x0f = jnp.floor(x)
        w1 = x - x0f
        w0 = 1.0 - w1
        x0 = x0f.astype(jnp.int32)
        x1 = x0 + 1
        m0 = (x0 >= 0) & (x0 <= LV - 1)
        m1 = (x1 >= 0) & (x1 <= LV - 1)
        awp = e[p] * inv
        swap = x0 == -1
        c_lo = jnp.where(swap, awp * w1, jnp.where(m0, awp * w0, 0.0))
        c_hi = jnp.where(swap, 0.0, jnp.where(m1, awp * w1, 0.0))
        lv0 = jnp.clip(x0, 0, LV - 1)
        idx_ref[:, p * 16:(p + 1) * 16] = lv0 * BZ + hoff
        coeff_ref[:, (2 * p) * 16:(2 * p + 1) * 16] = c_lo
        coeff_ref[:, (2 * p + 1) * 16:(2 * p + 2) * 16] = c_hi


def _prep(qf, w_cat, b_cat, refq, bm=512):
    M = qf.shape[0]
    grid = (M // bm,)
    return pl.pallas_call(
        functools.partial(_prep_kernel, bm),
        grid=grid,
        in_specs=[
            pl.BlockSpec((bm, EMBED_DIM), lambda i: (i, 0)),
            pl.BlockSpec((128, EMBED_DIM), lambda i: (0, 0)),
            pl.BlockSpec((1, 128), lambda i: (0, 0)),
            pl.BlockSpec((bm, 2), lambda i: (i, 0)),
        ],
        out_specs=[
            pl.BlockSpec((bm, 64), lambda i: (i, 0)),
            pl.BlockSpec((bm, 128), lambda i: (i, 0)),
        ],
        out_shape=[
            jax.ShapeDtypeStruct((M, 64), jnp.int32),
            jax.ShapeDtypeStruct((M, 128), jnp.float32),
        ],
    )(qf, w_cat, b_cat.reshape(1, 128), refq)


def _sc_gather_kernel(table_hbm, idx_hbm, coeff_hbm, out_hbm,
                      idx_v, coeff_v, rows_v, out_v, sem):
    wid = lax.axis_index("s") * NC + lax.axis_index("c")

    def splat(vec, h_full):
        return lax.gather(
            vec, h_full[:, None],
            lax.GatherDimensionNumbers(
                offset_dims=(), collapsed_slice_dims=(0,),
                start_index_map=(0,)),
            slice_sizes=(1,),
            mode=lax.GatherScatterMode.PROMISE_IN_BOUNDS)

    def chunk_body(g, carry):
        rb0 = wid * RB_PER_W + g * RB_PER_CHUNK
        pltpu.sync_copy(idx_hbm.at[pl.ds(rb0, RB_PER_CHUNK), :], idx_v)
        pltpu.sync_copy(coeff_hbm.at[pl.ds(rb0 * 128, CHUNK_E)], coeff_v)
        descs = []
        for i in range(RB_PER_CHUNK):
            descs.append(pltpu.async_copy(
                table_hbm.at[idx_v.at[i]],
                rows_v.at[pl.ds(i * 64, 64), :],
                sem,
            ))
        for dsc in descs:
            dsc.wait()

        def row_body(r, carry2):
            rb = r // NUM_HEADS
            h = r % NUM_HEADS
            h_full = jnp.full((L,), h, jnp.int32)
            acc = [jnp.zeros((L,), jnp.float32) for _ in range(4)]
            for p in range(4):
                pos = rb * 64 + p * 16 + h
                c_lo = splat(coeff_v[pl.ds(rb * 128 + p * 32, L)], h_full)
                c_hi = splat(coeff_v[pl.ds(rb * 128 + p * 32 + 16, L)], h_full)
                for j in range(4):
                    acc[j] = (acc[j]
                              + c_lo * rows_v[pos, pl.ds(j * 16, L)]
                              + c_hi * rows_v[pos, pl.ds(64 + j * 16, L)])
            for j in range(4):
                out_v[rb, pl.ds(h * HEAD_DIM + j * 16, L)] = acc[j]
            return carry2

        lax.fori_loop(0, CHUNK_R, row_body, 0, unroll=2)
        pltpu.sync_copy(out_v, out_hbm.at[pl.ds(rb0, RB_PER_CHUNK), :])
        return carry

    lax.fori_loop(0, CHUNKS_PER_W, chunk_body, 0)


@functools.cache
def _sc_gather_fn():
    return pl.kernel(
        _sc_gather_kernel,
        out_type=jax.ShapeDtypeStruct((ROWB, EMBED_DIM), jnp.float32),
        mesh=plsc.VectorSubcoreMesh(core_axis_name="c", subcore_axis_name="s",
                                    num_cores=NC, num_subcores=NS),
        scratch_types=[
            pltpu.VMEM((RB_PER_CHUNK, 64), jnp.int32),
            pltpu.VMEM((CHUNK_E,), jnp.float32),
            pltpu.VMEM((RB_PER_CHUNK * 64, 2 * HEAD_DIM), jnp.float32),
            pltpu.VMEM((RB_PER_CHUNK, EMBED_DIM), jnp.float32),
            pltpu.SemaphoreType.DMA,
        ],
    )


def _sc_gather(table, idx_all, coeff_all):
    return _sc_gather_fn()(table, idx_all, coeff_all.reshape(-1))


_PERM = [ (c % 16) * 4 + c // 16 for c in range(64) ]


def kernel(query, value, value_key_padding_mask, value_valid_ratio,
           reference_point, snippet_num, W_so, b_so, W_aw, b_aw, W_v, b_v,
           W_o, b_o):
    Lq, bz, d = query.shape
    Lv = value.shape[0]

    perm = jnp.array(_PERM, dtype=jnp.int32)
    w_cat = jnp.concatenate([W_so[perm], W_aw[perm]], axis=0)
    b_cat = jnp.concatenate([b_so[perm], b_aw[perm]])

    maskf = 1.0 - value_key_padding_mask.T.reshape(Lv * bz, 1).astype(jnp.float32)
    mm = _matmul_bias(value.reshape(Lv * bz, d), W_v, b_v, mask_col=maskf)
    table = _pair_build(mm)

    refq = jnp.transpose(reference_point, (1, 0, 2)).reshape(Lq * bz, 2)
    qf = query.reshape(Lq * bz, d)
    idx_all, coeff_all = _prep(qf, w_cat, b_cat, refq)

    attn = _sc_gather(table, idx_all, coeff_all)

    out = _matmul_bias(attn, W_o, b_o)
    return out.reshape(Lq, bz, d)


# speedup vs baseline: 1.8859x; 1.3896x over previous
import functools

import jax
import jax.numpy as jnp
from jax import lax
from jax.experimental import pallas as pl
from jax.experimental.pallas import tpu as pltpu
from jax.experimental.pallas import tpu_sc as plsc

EMBED_DIM = 1024
NUM_HEADS = 16
NUM_POINTS = 4
HEAD_DIM = EMBED_DIM // NUM_HEADS
LQ = 2048
LV = 2048
BZ = 4

NC, NS, L = 2, 16, 16
NW = NC * NS

R_TOTAL = LQ * BZ * NUM_HEADS
ROWB = LQ * BZ
ENTRIES = ROWB * 128
RB_PER_CHUNK = 4
CHUNK_E = RB_PER_CHUNK * 128
CHUNK_R = RB_PER_CHUNK * NUM_HEADS
RB_PER_W = ROWB // NW
CHUNKS_PER_W = RB_PER_W // RB_PER_CHUNK


def _mm(x, w):
    return lax.dot_general(x, w, (((1,), (1,)), ((), ())),
                           preferred_element_type=jnp.float32)


def _matmul_bias_kernel(x_ref, w_ref, b_ref, o_ref):
    o_ref[...] = _mm(x_ref[...], w_ref[...]) + b_ref[...]


def _matmul_bias_mask_kernel(x_ref, w_ref, b_ref, m_ref, o_ref):
    o_ref[...] = (_mm(x_ref[...], w_ref[...]) + b_ref[...]) * m_ref[...]


def _matmul_bias(x, w, b, mask_col=None, bm=512):
    M, K = x.shape
    N = w.shape[0]
    grid = (M // bm,)
    if mask_col is None:
        return pl.pallas_call(
            _matmul_bias_kernel,
            grid=grid,
            in_specs=[
                pl.BlockSpec((bm, K), lambda i: (i, 0)),
                pl.BlockSpec((N, K), lambda i: (0, 0)),
                pl.BlockSpec((1, N), lambda i: (0, 0)),
            ],
            out_specs=pl.BlockSpec((bm, N), lambda i: (i, 0)),
            out_shape=jax.ShapeDtypeStruct((M, N), jnp.float32),
        )(x, w, b.reshape(1, N))
    return pl.pallas_call(
        _matmul_bias_mask_kernel,
        grid=grid,
        in_specs=[
            pl.BlockSpec((bm, K), lambda i: (i, 0)),
            pl.BlockSpec((N, K), lambda i: (0, 0)),
            pl.BlockSpec((1, N), lambda i: (0, 0)),
            pl.BlockSpec((bm, 1), lambda i: (i, 0)),
        ],
        out_specs=pl.BlockSpec((bm, N), lambda i: (i, 0)),
        out_shape=jax.ShapeDtypeStruct((M, N), jnp.float32),
    )(x, w, b.reshape(1, N), mask_col)


def _pair_kernel(mm_ref, mmn_ref, o_ref):
    mm = mm_ref[...]
    for hh in range(2):
        lo = mm[:, hh * HEAD_DIM:(hh + 1) * HEAD_DIM]
        hi = jnp.concatenate(
            [mm[4:, hh * HEAD_DIM:(hh + 1) * HEAD_DIM],
             mmn_ref[0:4, hh * HEAD_DIM:(hh + 1) * HEAD_DIM]], axis=0)
        o_ref[hh, :, 0:HEAD_DIM] = lo
        o_ref[hh, :, HEAD_DIM:] = hi


def _pair_build(mm):
    nblk = LV // 128
    table3 = pl.pallas_call(
        _pair_kernel,
        grid=(NUM_HEADS // 2, nblk),
        in_specs=[
            pl.BlockSpec((512, 2 * HEAD_DIM), lambda hp, i: (i, hp)),
            pl.BlockSpec((8, 2 * HEAD_DIM),
                         lambda hp, i: (jnp.minimum(i + 1, nblk - 1) * 64, hp)),
        ],
        out_specs=pl.BlockSpec((2, 512, 2 * HEAD_DIM), lambda hp, i: (hp, i, 0)),
        out_shape=jax.ShapeDtypeStruct((NUM_HEADS, ROWB, 2 * HEAD_DIM),
                                       jnp.float32),
    )(mm, mm)
    return table3.reshape(R_TOTAL, 2 * HEAD_DIM)


def _prep_kernel(bm, q_ref, w_ref, b_ref, rp_ref, idx_ref, coeff_ref):
    soaw = _mm(q_ref[...], w_ref[...]) + b_ref[...]
    rp = rp_ref[...]
    ref_c = rp[:, 0:1]
    ref_w = rp[:, 1:2]
    i = pl.program_id(0)
    row = lax.broadcasted_iota(jnp.int32, (bm, 1), 0) + i * bm
    boff = row % BZ
    h_iota = lax.broadcasted_iota(jnp.int32, (bm, NUM_HEADS), 1)
    hoff = boff + h_iota * ROWB

    a = [soaw[:, 64 + p * 16:64 + (p + 1) * 16] for p in range(4)]
    mx = jnp.maximum(jnp.maximum(a[0], a[1]), jnp.maximum(a[2], a[3]))
    e = [jnp.exp(x - mx) for x in a]
    inv = 1.0 / (e[0] + e[1] + e[2] + e[3])

    for p in range(4):
        x = (ref_c + soaw[:, p * 16:(p + 1) * 16] * (ref_w * 0.125)) * float(LV - 1)
        x0f = jnp.floor(x)
        w1 = x - x0f
        w0 = 1.0 - w1
        x0 = x0f.astype(jnp.int32)
        x1 = x0 + 1
        m0 = (x0 >= 0) & (x0 <= LV - 1)
        m1 = (x1 >= 0) & (x1 <= LV - 1)
        awp = e[p] * inv
        swap = x0 == -1
        c_lo = jnp.where(swap, awp * w1, jnp.where(m0, awp * w0, 0.0))
        c_hi = jnp.where(swap, 0.0, jnp.where(m1, awp * w1, 0.0))
        lv0 = jnp.clip(x0, 0, LV - 1)
        idx_ref[:, p * 16:(p + 1) * 16] = lv0 * BZ + hoff
        coeff_ref[:, (2 * p) * 16:(2 * p + 1) * 16] = c_lo
        coeff_ref[:, (2 * p + 1) * 16:(2 * p + 2) * 16] = c_hi


def _prep(qf, w_cat, b_cat, refq, bm=512):
    M = qf.shape[0]
    grid = (M // bm,)
    return pl.pallas_call(
        functools.partial(_prep_kernel, bm),
        grid=grid,
        in_specs=[
            pl.BlockSpec((bm, EMBED_DIM), lambda i: (i, 0)),
            pl.BlockSpec((128, EMBED_DIM), lambda i: (0, 0)),
            pl.BlockSpec((1, 128), lambda i: (0, 0)),
            pl.BlockSpec((bm, 2), lambda i: (i, 0)),
        ],
        out_specs=[
            pl.BlockSpec((bm, 64), lambda i: (i, 0)),
            pl.BlockSpec((bm, 128), lambda i: (i, 0)),
        ],
        out_shape=[
            jax.ShapeDtypeStruct((M, 64), jnp.int32),
            jax.ShapeDtypeStruct((M, 128), jnp.float32),
        ],
    )(qf, w_cat, b_cat.reshape(1, 128), refq)


def _sc_gather_kernel(table_hbm, idx_hbm, coeff_hbm, out_hbm,
                      idx_v, coeff_v, rows_v, out_v,
                      sem_i0, sem_i1, sem_c0, sem_c1, sem_r, sem_o0, sem_o1):
    wid = lax.axis_index("s") * NC + lax.axis_index("c")
    rbw = wid * RB_PER_W
    NCH = CHUNKS_PER_W

    IDX = [idx_v.at[0], idx_v.at[1]]
    CO = [coeff_v.at[0], coeff_v.at[1]]
    RW = [rows_v.at[0], rows_v.at[1]]
    OU = [out_v.at[0], out_v.at[1]]
    SI = [sem_i0, sem_i1]
    SCm = [sem_c0, sem_c1]
    SO = [sem_o0, sem_o1]

    def idx_src(g):
        return idx_hbm.at[pl.ds(rbw + g * RB_PER_CHUNK, RB_PER_CHUNK), :]

    def coeff_src(g):
        return coeff_hbm.at[pl.ds((rbw + g * RB_PER_CHUNK) * 128, CHUNK_E)]

    def out_dst(g):
        return out_hbm.at[pl.ds(rbw + g * RB_PER_CHUNK, RB_PER_CHUNK), :]

    def fire_gather(s):
        for i in range(RB_PER_CHUNK):
            pltpu.async_copy(table_hbm.at[IDX[s].at[i]],
                             RW[s].at[pl.ds(i * 64, 64), :], sem_r)

    def drain_gather(s):
        for i in range(RB_PER_CHUNK):
            pltpu.make_async_copy(table_hbm.at[pl.ds(0, 64), :],
                                  RW[s].at[pl.ds(i * 64, 64), :], sem_r).wait()

    def splat(vec, h_full):
        return lax.gather(
            vec, h_full[:, None],
            lax.GatherDimensionNumbers(
                offset_dims=(), collapsed_slice_dims=(0,),
                start_index_map=(0,)),
            slice_sizes=(1,),
            mode=lax.GatherScatterMode.PROMISE_IN_BOUNDS)

    def compute(s):
        co, rw, ou = CO[s], RW[s], OU[s]

        def row_body(r, carry2):
            rb = r // NUM_HEADS
            h = r % NUM_HEADS
            h_full = jnp.full((L,), h, jnp.int32)
            acc = [jnp.zeros((L,), jnp.float32) for _ in range(4)]
            for p in range(4):
                pos = rb * 64 + p * 16 + h
                c_lo = splat(co[pl.ds(rb * 128 + p * 32, L)], h_full)
                c_hi = splat(co[pl.ds(rb * 128 + p * 32 + 16, L)], h_full)
                for j in range(4):
                    acc[j] = (acc[j]
                              + c_lo * rw[pos, pl.ds(j * 16, L)]
                              + c_hi * rw[pos, pl.ds(64 + j * 16, L)])
            for j in range(4):
                ou[rb, pl.ds(h * HEAD_DIM + j * 16, L)] = acc[j]
            return carry2

        lax.fori_loop(0, CHUNK_R, row_body, 0, unroll=2)

    def body(g, s):
        t = 1 - s
        drain_gather(s)

        @pl.when(g < NCH - 1)
        def _():
            pltpu.make_async_copy(idx_src(0), IDX[t], SI[t]).wait()
            fire_gather(t)

        @pl.when(g < NCH - 2)
        def _():
            pltpu.async_copy(idx_src(g + 2), IDX[s], SI[s])

        @pl.when(g >= 1)
        def _():
            pltpu.make_async_copy(coeff_src(0), CO[s], SCm[s]).wait()

        @pl.when(g >= 2)
        def _():
            pltpu.make_async_copy(out_dst(0), OU[s], SO[s]).wait()

        compute(s)
        pltpu.async_copy(OU[s], out_dst(g), SO[s])

        @pl.when(g < NCH - 2)
        def _():
            pltpu.async_copy(coeff_src(g + 2), CO[s], SCm[s])

    pltpu.sync_copy(idx_src(0), IDX[0])
    pltpu.sync_copy(coeff_src(0), CO[0])
    fire_gather(0)
    pltpu.async_copy(idx_src(1), IDX[1], SI[1])
    pltpu.async_copy(coeff_src(1), CO[1], SCm[1])

    def loop_body(gg, carry):
        body(2 * gg, 0)
        body(2 * gg + 1, 1)
        return carry

    lax.fori_loop(0, NCH // 2, loop_body, 0)
    pltpu.make_async_copy(out_dst(0), OU[0], SO[0]).wait()
    pltpu.make_async_copy(out_dst(0), OU[1], SO[1]).wait()


@functools.cache
def _sc_gather_fn():
    return pl.kernel(
        _sc_gather_kernel,
        out_type=jax.ShapeDtypeStruct((ROWB, EMBED_DIM), jnp.float32),
        mesh=plsc.VectorSubcoreMesh(core_axis_name="c", subcore_axis_name="s",
                                    num_cores=NC, num_subcores=NS),
        scratch_types=[
            pltpu.VMEM((2, RB_PER_CHUNK, 64), jnp.int32),
            pltpu.VMEM((2, CHUNK_E), jnp.float32),
            pltpu.VMEM((2, RB_PER_CHUNK * 64, 2 * HEAD_DIM), jnp.float32),
            pltpu.VMEM((2, RB_PER_CHUNK, EMBED_DIM), jnp.float32),
            pltpu.SemaphoreType.DMA,
            pltpu.SemaphoreType.DMA,
            pltpu.SemaphoreType.DMA,
            pltpu.SemaphoreType.DMA,
            pltpu.SemaphoreType.DMA,
            pltpu.SemaphoreType.DMA,
            pltpu.SemaphoreType.DMA,
        ],
    )


def _sc_gather(table, idx_all, coeff_all):
    return _sc_gather_fn()(table, idx_all, coeff_all.reshape(-1))


_PERM = [ (c % 16) * 4 + c // 16 for c in range(64) ]


def kernel(query, value, value_key_padding_mask, value_valid_ratio,
           reference_point, snippet_num, W_so, b_so, W_aw, b_aw, W_v, b_v,
           W_o, b_o):
    Lq, bz, d = query.shape
    Lv = value.shape[0]

    perm = jnp.array(_PERM, dtype=jnp.int32)
    w_cat = jnp.concatenate([W_so[perm], W_aw[perm]], axis=0)
    b_cat = jnp.concatenate([b_so[perm], b_aw[perm]])

    maskf = 1.0 - value_key_padding_mask.T.reshape(Lv * bz, 1).astype(jnp.float32)
    mm = _matmul_bias(value.reshape(Lv * bz, d), W_v, b_v, mask_col=maskf)
    table = _pair_build(mm)

    refq = jnp.transpose(reference_point, (1, 0, 2)).reshape(Lq * bz, 2)
    qf = query.reshape(Lq * bz, d)
    idx_all, coeff_all = _prep(qf, w_cat, b_cat, refq)

    attn = _sc_gather(table, idx_all, coeff_all)

    out = _matmul_bias(attn, W_o, b_o)
    return out.reshape(Lq, bz, d)


# speedup vs baseline: 2.2720x; 1.2047x over previous
import functools

import jax
import jax.numpy as jnp
from jax import lax
from jax.experimental import pallas as pl
from jax.experimental.pallas import tpu as pltpu
from jax.experimental.pallas import tpu_sc as plsc

EMBED_DIM = 1024
NUM_HEADS = 16
NUM_POINTS = 4
HEAD_DIM = EMBED_DIM // NUM_HEADS
LQ = 2048
LV = 2048
BZ = 4

NC, NS, L = 2, 16, 16
NW = NC * NS

R_TOTAL = LQ * BZ * NUM_HEADS
ROWB = LQ * BZ
ENTRIES = ROWB * 128
RB_PER_CHUNK = 4
CHUNK_E = RB_PER_CHUNK * 128
CHUNK_R = RB_PER_CHUNK * NUM_HEADS
RB_PER_W = ROWB // NW
CHUNKS_PER_W = RB_PER_W // RB_PER_CHUNK


def _mm(x, w):
    return lax.dot_general(x, w, (((1,), (1,)), ((), ())),
                           preferred_element_type=jnp.float32)


def _matmul_bias_kernel(x_ref, w_ref, b_ref, o_ref):
    o_ref[...] = _mm(x_ref[...], w_ref[...]) + b_ref[...]


def _matmul_bias_mask_kernel(x_ref, w_ref, b_ref, m_ref, o_ref):
    o_ref[...] = (_mm(x_ref[...], w_ref[...]) + b_ref[...]) * m_ref[...]


def _matmul_bias(x, w, b, mask_col=None, bm=512):
    M, K = x.shape
    N = w.shape[0]
    grid = (M // bm,)
    if mask_col is None:
        return pl.pallas_call(
            _matmul_bias_kernel,
            grid=grid,
            in_specs=[
                pl.BlockSpec((bm, K), lambda i: (i, 0)),
                pl.BlockSpec((N, K), lambda i: (0, 0)),
                pl.BlockSpec((1, N), lambda i: (0, 0)),
            ],
            out_specs=pl.BlockSpec((bm, N), lambda i: (i, 0)),
            out_shape=jax.ShapeDtypeStruct((M, N), jnp.float32),
        )(x, w, b.reshape(1, N))
    return pl.pallas_call(
        _matmul_bias_mask_kernel,
        grid=grid,
        in_specs=[
            pl.BlockSpec((bm, K), lambda i: (i, 0)),
            pl.BlockSpec((N, K), lambda i: (0, 0)),
            pl.BlockSpec((1, N), lambda i: (0, 0)),
            pl.BlockSpec((bm, 1), lambda i: (i, 0)),
        ],
        out_specs=pl.BlockSpec((bm, N), lambda i: (i, 0)),
        out_shape=jax.ShapeDtypeStruct((M, N), jnp.float32),
    )(x, w, b.reshape(1, N), mask_col)


def _vproj_pair_kernel(x_ref, xh_ref, w_ref, b_ref, m_ref, mh_ref, o_ref):
    mm = (_mm(x_ref[...], w_ref[...]) + b_ref[...]) * m_ref[...]
    mmh = (_mm(xh_ref[...], w_ref[...]) + b_ref[...]) * mh_ref[...]
    hi = jnp.concatenate([mm[4:], mmh[0:4]], axis=0)
    for h in range(NUM_HEADS):
        o_ref[h, :, 0:HEAD_DIM] = mm[:, h * HEAD_DIM:(h + 1) * HEAD_DIM]
        o_ref[h, :, HEAD_DIM:] = hi[:, h * HEAD_DIM:(h + 1) * HEAD_DIM]


def _vproj_pair(x, w, b, maskf):
    nblk = LV // 128
    table3 = pl.pallas_call(
        _vproj_pair_kernel,
        grid=(nblk,),
        in_specs=[
            pl.BlockSpec((512, EMBED_DIM), lambda i: (i, 0)),
            pl.BlockSpec((8, EMBED_DIM),
                         lambda i: (jnp.minimum(i + 1, nblk - 1) * 64, 0)),
            pl.BlockSpec((EMBED_DIM, EMBED_DIM), lambda i: (0, 0)),
            pl.BlockSpec((1, EMBED_DIM), lambda i: (0, 0)),
            pl.BlockSpec((512, 1), lambda i: (i, 0)),
            pl.BlockSpec((8, 1), lambda i: (jnp.minimum(i + 1, nblk - 1) * 64, 0)),
        ],
        out_specs=pl.BlockSpec((NUM_HEADS, 512, 2 * HEAD_DIM),
                               lambda i: (0, i, 0)),
        out_shape=jax.ShapeDtypeStruct((NUM_HEADS, ROWB, 2 * HEAD_DIM),
                                       jnp.float32),
    )(x, x, w, b.reshape(1, EMBED_DIM), maskf, maskf)
    return table3.reshape(R_TOTAL, 2 * HEAD_DIM)


def _prep_kernel(bm, q_ref, w_ref, b_ref, rp_ref, idx_ref, coeff_ref):
    soaw = _mm(q_ref[...], w_ref[...]) + b_ref[...]
    rp = rp_ref[...]
    ref_c = rp[:, 0:1]
    ref_w = rp[:, 1:2]
    i = pl.program_id(0)
    row = lax.broadcasted_iota(jnp.int32, (bm, 1), 0) + i * bm
    boff = row % BZ
    h_iota = lax.broadcasted_iota(jnp.int32, (bm, NUM_HEADS), 1)
    hoff = boff + h_iota * ROWB

    a = [soaw[:, 64 + p * 16:64 + (p + 1) * 16] for p in range(4)]
    mx = jnp.maximum(jnp.maximum(a[0], a[1]), jnp.maximum(a[2], a[3]))
    e = [jnp.exp(x - mx) for x in a]
    inv = 1.0 / (e[0] + e[1] + e[2] + e[3])

    for p in range(4):
        x = (ref_c + soaw[:, p * 16:(p + 1) * 16] * (ref_w * 0.125)) * float(LV - 1)
        x0f = jnp.floor(x)
        w1 = x - x0f
        w0 = 1.0 - w1
        x0 = x0f.astype(jnp.int32)
        x1 = x0 + 1
        m0 = (x0 >= 0) & (x0 <= LV - 1)
        m1 = (x1 >= 0) & (x1 <= LV - 1)
        awp = e[p] * inv
        swap = x0 == -1
        c_lo = jnp.where(swap, awp * w1, jnp.where(m0, awp * w0, 0.0))
        c_hi = jnp.where(swap, 0.0, jnp.where(m1, awp * w1, 0.0))
        lv0 = jnp.clip(x0, 0, LV - 1)
        idx_ref[:, p * 16:(p + 1) * 16] = lv0 * BZ + hoff
        coeff_ref[:, (2 * p) * 16:(2 * p + 1) * 16] = c_lo
        coeff_ref[:, (2 * p + 1) * 16:(2 * p + 2) * 16] = c_hi


def _prep(qf, w_cat, b_cat, refq, bm=512):
    M = qf.shape[0]
    grid = (M // bm,)
    return pl.pallas_call(
        functools.partial(_prep_kernel, bm),
        grid=grid,
        in_specs=[
            pl.BlockSpec((bm, EMBED_DIM), lambda i: (i, 0)),
            pl.BlockSpec((128, EMBED_DIM), lambda i: (0, 0)),
            pl.BlockSpec((1, 128), lambda i: (0, 0)),
            pl.BlockSpec((bm, 2), lambda i: (i, 0)),
        ],
        out_specs=[
            pl.BlockSpec((bm, 64), lambda i: (i, 0)),
            pl.BlockSpec((bm, 128), lambda i: (i, 0)),
        ],
        out_shape=[
            jax.ShapeDtypeStruct((M, 64), jnp.int32),
            jax.ShapeDtypeStruct((M, 128), jnp.float32),
        ],
    )(qf, w_cat, b_cat.reshape(1, 128), refq)


def _sc_gather_kernel(table_hbm, idx_hbm, coeff_hbm, out_hbm,
                      idx_v, coeff_v, rows_v, out_v,
                      sem_i0, sem_i1, sem_c0, sem_c1, sem_r, sem_o0, sem_o1):
    wid = lax.axis_index("s") * NC + lax.axis_index("c")
    rbw = wid * RB_PER_W
    NCH = CHUNKS_PER_W

    IDX = [idx_v.at[0], idx_v.at[1]]
    CO = [coeff_v.at[0], coeff_v.at[1]]
    RW = [rows_v.at[0], rows_v.at[1]]
    OU = [out_v.at[0], out_v.at[1]]
    SI = [sem_i0, sem_i1]
    SCm = [sem_c0, sem_c1]
    SO = [sem_o0, sem_o1]

    def idx_src(g):
        return idx_hbm.at[pl.ds(rbw + g * RB_PER_CHUNK, RB_PER_CHUNK), :]

    def coeff_src(g):
        return coeff_hbm.at[pl.ds((rbw + g * RB_PER_CHUNK) * 128, CHUNK_E)]

    def out_dst(g):
        return out_hbm.at[pl.ds(rbw + g * RB_PER_CHUNK, RB_PER_CHUNK), :]

    def fire_gather(s):
        for i in range(RB_PER_CHUNK):
            pltpu.async_copy(table_hbm.at[IDX[s].at[i]],
                             RW[s].at[pl.ds(i * 64, 64), :], sem_r)

    def drain_gather(s):
        for i in range(RB_PER_CHUNK):
            pltpu.make_async_copy(table_hbm.at[pl.ds(0, 64), :],
                                  RW[s].at[pl.ds(i * 64, 64), :], sem_r).wait()

    def splat(vec, h_full):
        return lax.gather(
            vec, h_full[:, None],
            lax.GatherDimensionNumbers(
                offset_dims=(), collapsed_slice_dims=(0,),
                start_index_map=(0,)),
            slice_sizes=(1,),
            mode=lax.GatherScatterMode.PROMISE_IN_BOUNDS)

    def compute(s):
        co, rw, ou = CO[s], RW[s], OU[s]

        def row_body(r, carry2):
            rb = r // NUM_HEADS
            h = r % NUM_HEADS
            h_full = jnp.full((L,), h, jnp.int32)
            acc = [jnp.zeros((L,), jnp.float32) for _ in range(4)]
            for p in range(4):
                pos = rb * 64 + p * 16 + h
                c_lo = splat(co[pl.ds(rb * 128 + p * 32, L)], h_full)
                c_hi = splat(co[pl.ds(rb * 128 + p * 32 + 16, L)], h_full)
                for j in range(4):
                    acc[j] = (acc[j]
                              + c_lo * rw[pos, pl.ds(j * 16, L)]
                              + c_hi * rw[pos, pl.ds(64 + j * 16, L)])
            for j in range(4):
                ou[rb, pl.ds(h * HEAD_DIM + j * 16, L)] = acc[j]
            return carry2

        lax.fori_loop(0, CHUNK_R, row_body, 0, unroll=2)

    def body(g, s):
        t = 1 - s
        drain_gather(s)

        @pl.when(g < NCH - 1)
        def _():
            pltpu.make_async_copy(idx_src(0), IDX[t], SI[t]).wait()
            fire_gather(t)

        @pl.when(g < NCH - 2)
        def _():
            pltpu.async_copy(idx_src(g + 2), IDX[s], SI[s])

        @pl.when(g >= 1)
        def _():
            pltpu.make_async_copy(coeff_src(0), CO[s], SCm[s]).wait()

        @pl.when(g >= 2)
        def _():
            pltpu.make_async_copy(out_dst(0), OU[s], SO[s]).wait()

        compute(s)
        pltpu.async_copy(OU[s], out_dst(g), SO[s])

        @pl.when(g < NCH - 2)
        def _():
            pltpu.async_copy(coeff_src(g + 2), CO[s], SCm[s])

    pltpu.sync_copy(idx_src(0), IDX[0])
    pltpu.sync_copy(coeff_src(0), CO[0])
    fire_gather(0)
    pltpu.async_copy(idx_src(1), IDX[1], SI[1])
    pltpu.async_copy(coeff_src(1), CO[1], SCm[1])

    def loop_body(gg, carry):
        body(2 * gg, 0)
        body(2 * gg + 1, 1)
        return carry

    lax.fori_loop(0, NCH // 2, loop_body, 0)
    pltpu.make_async_copy(out_dst(0), OU[0], SO[0]).wait()
    pltpu.make_async_copy(out_dst(0), OU[1], SO[1]).wait()


@functools.cache
def _sc_gather_fn():
    return pl.kernel(
        _sc_gather_kernel,
        out_type=jax.ShapeDtypeStruct((ROWB, EMBED_DIM), jnp.float32),
        mesh=plsc.VectorSubcoreMesh(core_axis_name="c", subcore_axis_name="s",
                                    num_cores=NC, num_subcores=NS),
        scratch_types=[
            pltpu.VMEM((2, RB_PER_CHUNK, 64), jnp.int32),
            pltpu.VMEM((2, CHUNK_E), jnp.float32),
            pltpu.VMEM((2, RB_PER_CHUNK * 64, 2 * HEAD_DIM), jnp.float32),
            pltpu.VMEM((2, RB_PER_CHUNK, EMBED_DIM), jnp.float32),
            pltpu.SemaphoreType.DMA,
            pltpu.SemaphoreType.DMA,
            pltpu.SemaphoreType.DMA,
            pltpu.SemaphoreType.DMA,
            pltpu.SemaphoreType.DMA,
            pltpu.SemaphoreType.DMA,
            pltpu.SemaphoreType.DMA,
        ],
    )


def _sc_gather(table, idx_all, coeff_all):
    return _sc_gather_fn()(table, idx_all, coeff_all.reshape(-1))


_PERM = [ (c % 16) * 4 + c // 16 for c in range(64) ]


def kernel(query, value, value_key_padding_mask, value_valid_ratio,
           reference_point, snippet_num, W_so, b_so, W_aw, b_aw, W_v, b_v,
           W_o, b_o):
    Lq, bz, d = query.shape
    Lv = value.shape[0]

    perm = jnp.array(_PERM, dtype=jnp.int32)
    w_cat = jnp.concatenate([W_so[perm], W_aw[perm]], axis=0)
    b_cat = jnp.concatenate([b_so[perm], b_aw[perm]])

    maskf = 1.0 - value_key_padding_mask.T.reshape(Lv * bz, 1).astype(jnp.float32)
    table = _vproj_pair(value.reshape(Lv * bz, d), W_v, b_v, maskf)

    refq = jnp.transpose(reference_point, (1, 0, 2)).reshape(Lq * bz, 2)
    qf = query.reshape(Lq * bz, d)
    idx_all, coeff_all = _prep(qf, w_cat, b_cat, refq)

    attn = _sc_gather(table, idx_all, coeff_all)

    out = _matmul_bias(attn, W_o, b_o)
    return out.reshape(Lq, bz, d)


# speedup vs baseline: 3.1533x; 1.3879x over previous
import functools

import jax
import jax.numpy as jnp
from jax import lax
from jax.experimental import pallas as pl
from jax.experimental.pallas import tpu as pltpu
from jax.experimental.pallas import tpu_sc as plsc

EMBED_DIM = 1024
NUM_HEADS = 16
NUM_POINTS = 4
HEAD_DIM = EMBED_DIM // NUM_HEADS
LQ = 2048
LV = 2048
BZ = 4

NC, NS, L = 2, 16, 16
NW = NC * NS

R_TOTAL = LQ * BZ * NUM_HEADS
ROWB = LQ * BZ
ENTRIES = ROWB * 128
RB_PER_CHUNK = 4
CHUNK_E = RB_PER_CHUNK * 128
CHUNK_R = RB_PER_CHUNK * NUM_HEADS
RB_PER_W = ROWB // NW
CHUNKS_PER_W = RB_PER_W // RB_PER_CHUNK


def _mm(x, w):
    return lax.dot_general(x, w, (((1,), (1,)), ((), ())),
                           preferred_element_type=jnp.float32)


def _matmul_bias_kernel(x_ref, w_ref, b_ref, o_ref):
    o_ref[...] = _mm(x_ref[...], w_ref[...]) + b_ref[...]


def _matmul_bias_mask_kernel(x_ref, w_ref, b_ref, m_ref, o_ref):
    o_ref[...] = (_mm(x_ref[...], w_ref[...]) + b_ref[...]) * m_ref[...]


def _matmul_bias(x, w, b, mask_col=None, bm=512):
    M, K = x.shape
    N = w.shape[0]
    grid = (M // bm,)
    if mask_col is None:
        return pl.pallas_call(
            _matmul_bias_kernel,
            grid=grid,
            in_specs=[
                pl.BlockSpec((bm, K), lambda i: (i, 0)),
                pl.BlockSpec((N, K), lambda i: (0, 0)),
                pl.BlockSpec((1, N), lambda i: (0, 0)),
            ],
            out_specs=pl.BlockSpec((bm, N), lambda i: (i, 0)),
            out_shape=jax.ShapeDtypeStruct((M, N), jnp.float32),
        )(x, w, b.reshape(1, N))
    return pl.pallas_call(
        _matmul_bias_mask_kernel,
        grid=grid,
        in_specs=[
            pl.BlockSpec((bm, K), lambda i: (i, 0)),
            pl.BlockSpec((N, K), lambda i: (0, 0)),
            pl.BlockSpec((1, N), lambda i: (0, 0)),
            pl.BlockSpec((bm, 1), lambda i: (i, 0)),
        ],
        out_specs=pl.BlockSpec((bm, N), lambda i: (i, 0)),
        out_shape=jax.ShapeDtypeStruct((M, N), jnp.float32),
    )(x, w, b.reshape(1, N), mask_col)


def _vproj_pair_kernel3(x_ref, xh_ref, w_ref, b_ref, m_ref, mh_ref, o_ref):
    x = x_ref[...].reshape(512, EMBED_DIM)
    xh = xh_ref[...].reshape(8, EMBED_DIM)
    mm = (_mm(x, w_ref[...]) + b_ref[...]) * m_ref[...]
    mmh = (_mm(xh, w_ref[...]) + b_ref[...]) * mh_ref[...]
    hi = jnp.concatenate([mm[4:], mmh[0:4]], axis=0)
    for h in range(NUM_HEADS):
        o_ref[h, :, 0:HEAD_DIM] = mm[:, h * HEAD_DIM:(h + 1) * HEAD_DIM]
        o_ref[h, :, HEAD_DIM:] = hi[:, h * HEAD_DIM:(h + 1) * HEAD_DIM]


def _vproj_pair3(value, w, b, maskf):
    nblk = LV // 128
    table3 = pl.pallas_call(
        _vproj_pair_kernel3,
        grid=(nblk,),
        in_specs=[
            pl.BlockSpec((128, BZ, EMBED_DIM), lambda i: (i, 0, 0)),
            pl.BlockSpec((2, BZ, EMBED_DIM),
                         lambda i: (jnp.minimum(i + 1, nblk - 1) * 64, 0, 0)),
            pl.BlockSpec((EMBED_DIM, EMBED_DIM), lambda i: (0, 0)),
            pl.BlockSpec((1, EMBED_DIM), lambda i: (0, 0)),
            pl.BlockSpec((512, 1), lambda i: (i, 0)),
            pl.BlockSpec((8, 1), lambda i: (jnp.minimum(i + 1, nblk - 1) * 64, 0)),
        ],
        out_specs=pl.BlockSpec((NUM_HEADS, 512, 2 * HEAD_DIM),
                               lambda i: (0, i, 0)),
        out_shape=jax.ShapeDtypeStruct((NUM_HEADS, ROWB, 2 * HEAD_DIM),
                                       jnp.float32),
    )(value, value, w, b.reshape(1, EMBED_DIM), maskf, maskf)
    return table3.reshape(R_TOTAL, 2 * HEAD_DIM)


def _vproj_pair_kernel(x_ref, xh_ref, w_ref, b_ref, m_ref, mh_ref, o_ref):
    mm = (_mm(x_ref[...], w_ref[...]) + b_ref[...]) * m_ref[...]
    mmh = (_mm(xh_ref[...], w_ref[...]) + b_ref[...]) * mh_ref[...]
    hi = jnp.concatenate([mm[4:], mmh[0:4]], axis=0)
    for h in range(NUM_HEADS):
        o_ref[h, :, 0:HEAD_DIM] = mm[:, h * HEAD_DIM:(h + 1) * HEAD_DIM]
        o_ref[h, :, HEAD_DIM:] = hi[:, h * HEAD_DIM:(h + 1) * HEAD_DIM]


def _vproj_pair(x, w, b, maskf):
    nblk = LV // 128
    table3 = pl.pallas_call(
        _vproj_pair_kernel,
        grid=(nblk,),
        in_specs=[
            pl.BlockSpec((512, EMBED_DIM), lambda i: (i, 0)),
            pl.BlockSpec((8, EMBED_DIM),
                         lambda i: (jnp.minimum(i + 1, nblk - 1) * 64, 0)),
            pl.BlockSpec((EMBED_DIM, EMBED_DIM), lambda i: (0, 0)),
            pl.BlockSpec((1, EMBED_DIM), lambda i: (0, 0)),
            pl.BlockSpec((512, 1), lambda i: (i, 0)),
            pl.BlockSpec((8, 1), lambda i: (jnp.minimum(i + 1, nblk - 1) * 64, 0)),
        ],
        out_specs=pl.BlockSpec((NUM_HEADS, 512, 2 * HEAD_DIM),
                               lambda i: (0, i, 0)),
        out_shape=jax.ShapeDtypeStruct((NUM_HEADS, ROWB, 2 * HEAD_DIM),
                                       jnp.float32),
    )(x, x, w, b.reshape(1, EMBED_DIM), maskf, maskf)
    return table3.reshape(R_TOTAL, 2 * HEAD_DIM)


def _prep_kernel(bm, q_ref, w_ref, b_ref, rp_ref, idx_ref, coeff_ref):
    soaw = _mm(q_ref[...].reshape(bm, EMBED_DIM), w_ref[...]) + b_ref[...]
    rp = rp_ref[...]
    ref_c = rp[:, 0:1]
    ref_w = rp[:, 1:2]
    i = pl.program_id(0)
    row = lax.broadcasted_iota(jnp.int32, (bm, 1), 0) + i * bm
    boff = row % BZ
    h_iota = lax.broadcasted_iota(jnp.int32, (bm, NUM_HEADS), 1)
    hoff = boff + h_iota * ROWB

    a = [soaw[:, 64 + p * 16:64 + (p + 1) * 16] for p in range(4)]
    mx = jnp.maximum(jnp.maximum(a[0], a[1]), jnp.maximum(a[2], a[3]))
    e = [jnp.exp(x - mx) for x in a]
    inv = 1.0 / (e[0] + e[1] + e[2] + e[3])

    for p in range(4):
        x = (ref_c + soaw[:, p * 16:(p + 1) * 16] * (ref_w * 0.125)) * float(LV - 1)
        x0f = jnp.floor(x)
        w1 = x - x0f
        w0 = 1.0 - w1
        x0 = x0f.astype(jnp.int32)
        x1 = x0 + 1
        m0 = (x0 >= 0) & (x0 <= LV - 1)
        m1 = (x1 >= 0) & (x1 <= LV - 1)
        awp = e[p] * inv
        swap = x0 == -1
        c_lo = jnp.where(swap, awp * w1, jnp.where(m0, awp * w0, 0.0))
        c_hi = jnp.where(swap, 0.0, jnp.where(m1, awp * w1, 0.0))
        lv0 = jnp.clip(x0, 0, LV - 1)
        idx_ref[:, p * 16:(p + 1) * 16] = lv0 * BZ + hoff
        coeff_ref[:, (2 * p) * 16:(2 * p + 1) * 16] = c_lo
        coeff_ref[:, (2 * p + 1) * 16:(2 * p + 2) * 16] = c_hi


def _prep(qf, w_cat, b_cat, refq, bm=512):
    M = qf.shape[0] * qf.shape[1]
    grid = (M // bm,)
    return pl.pallas_call(
        functools.partial(_prep_kernel, bm),
        grid=grid,
        in_specs=[
            pl.BlockSpec((bm // BZ, BZ, EMBED_DIM), lambda i: (i, 0, 0)),
            pl.BlockSpec((128, EMBED_DIM), lambda i: (0, 0)),
            pl.BlockSpec((1, 128), lambda i: (0, 0)),
            pl.BlockSpec((bm, 2), lambda i: (i, 0)),
        ],
        out_specs=[
            pl.BlockSpec((bm, 64), lambda i: (i, 0)),
            pl.BlockSpec((bm, 128), lambda i: (i, 0)),
        ],
        out_shape=[
            jax.ShapeDtypeStruct((M, 64), jnp.int32),
            jax.ShapeDtypeStruct((M, 128), jnp.float32),
        ],
    )(qf, w_cat, b_cat.reshape(1, 128), refq)


def _sc_gather_kernel(table_hbm, idx_hbm, coeff_hbm, out_hbm,
                      idx_v, coeff_v, rows_v, out_v,
                      sem_i0, sem_i1, sem_c0, sem_c1, sem_r, sem_o0, sem_o1):
    wid = lax.axis_index("s") * NC + lax.axis_index("c")
    rbw = wid * RB_PER_W
    NCH = CHUNKS_PER_W

    IDX = [idx_v.at[0], idx_v.at[1]]
    CO = [coeff_v.at[0], coeff_v.at[1]]
    RW = [rows_v.at[0], rows_v.at[1]]
    OU = [out_v.at[0], out_v.at[1]]
    SI = [sem_i0, sem_i1]
    SCm = [sem_c0, sem_c1]
    SO = [sem_o0, sem_o1]

    def idx_src(g):
        return idx_hbm.at[pl.ds(rbw + g * RB_PER_CHUNK, RB_PER_CHUNK), :]

    def coeff_src(g):
        return coeff_hbm.at[pl.ds((rbw + g * RB_PER_CHUNK) * 128, CHUNK_E)]

    def out_dst(g):
        return out_hbm.at[pl.ds(rbw + g * RB_PER_CHUNK, RB_PER_CHUNK), :]

    def fire_gather(s):
        for i in range(RB_PER_CHUNK):
            pltpu.async_copy(table_hbm.at[IDX[s].at[i]],
                             RW[s].at[pl.ds(i * 64, 64), :], sem_r)

    def drain_gather(s):
        for i in range(RB_PER_CHUNK):
            pltpu.make_async_copy(table_hbm.at[pl.ds(0, 64), :],
                                  RW[s].at[pl.ds(i * 64, 64), :], sem_r).wait()

    def splat(vec, h_full):
        return lax.gather(
            vec, h_full[:, None],
            lax.GatherDimensionNumbers(
                offset_dims=(), collapsed_slice_dims=(0,),
                start_index_map=(0,)),
            slice_sizes=(1,),
            mode=lax.GatherScatterMode.PROMISE_IN_BOUNDS)

    def compute(s):
        co, rw, ou = CO[s], RW[s], OU[s]

        def row_body(r, carry2):
            rb = r // NUM_HEADS
            h = r % NUM_HEADS
            h_full = jnp.full((L,), h, jnp.int32)
            acc = [jnp.zeros((L,), jnp.float32) for _ in range(4)]
            for p in range(4):
                pos = rb * 64 + p * 16 + h
                c_lo = splat(co[pl.ds(rb * 128 + p * 32, L)], h_full)
                c_hi = splat(co[pl.ds(rb * 128 + p * 32 + 16, L)], h_full)
                for j in range(4):
                    acc[j] = (acc[j]
                              + c_lo * rw[pos, pl.ds(j * 16, L)]
                              + c_hi * rw[pos, pl.ds(64 + j * 16, L)])
            for j in range(4):
                ou[rb, pl.ds(h * HEAD_DIM + j * 16, L)] = acc[j]
            return carry2

        lax.fori_loop(0, CHUNK_R, row_body, 0, unroll=2)

    def body(g, s):
        t = 1 - s
        drain_gather(s)

        @pl.when(g < NCH - 1)
        def _():
            pltpu.make_async_copy(idx_src(0), IDX[t], SI[t]).wait()
            fire_gather(t)

        @pl.when(g < NCH - 2)
        def _():
            pltpu.async_copy(idx_src(g + 2), IDX[s], SI[s])

        @pl.when(g >= 1)
        def _():
            pltpu.make_async_copy(coeff_src(0), CO[s], SCm[s]).wait()

        @pl.when(g >= 2)
        def _():
            pltpu.make_async_copy(out_dst(0), OU[s], SO[s]).wait()

        compute(s)
        pltpu.async_copy(OU[s], out_dst(g), SO[s])

        @pl.when(g < NCH - 2)
        def _():
            pltpu.async_copy(coeff_src(g + 2), CO[s], SCm[s])

    pltpu.sync_copy(idx_src(0), IDX[0])
    pltpu.sync_copy(coeff_src(0), CO[0])
    fire_gather(0)
    pltpu.async_copy(idx_src(1), IDX[1], SI[1])
    pltpu.async_copy(coeff_src(1), CO[1], SCm[1])

    def loop_body(gg, carry):
        body(2 * gg, 0)
        body(2 * gg + 1, 1)
        return carry

    lax.fori_loop(0, NCH // 2, loop_body, 0)
    pltpu.make_async_copy(out_dst(0), OU[0], SO[0]).wait()
    pltpu.make_async_copy(out_dst(0), OU[1], SO[1]).wait()


@functools.cache
def _sc_gather_fn():
    return pl.kernel(
        _sc_gather_kernel,
        out_type=jax.ShapeDtypeStruct((ROWB, EMBED_DIM), jnp.float32),
        mesh=plsc.VectorSubcoreMesh(core_axis_name="c", subcore_axis_name="s",
                                    num_cores=NC, num_subcores=NS),
        scratch_types=[
            pltpu.VMEM((2, RB_PER_CHUNK, 64), jnp.int32),
            pltpu.VMEM((2, CHUNK_E), jnp.float32),
            pltpu.VMEM((2, RB_PER_CHUNK * 64, 2 * HEAD_DIM), jnp.float32),
            pltpu.VMEM((2, RB_PER_CHUNK, EMBED_DIM), jnp.float32),
            pltpu.SemaphoreType.DMA,
            pltpu.SemaphoreType.DMA,
            pltpu.SemaphoreType.DMA,
            pltpu.SemaphoreType.DMA,
            pltpu.SemaphoreType.DMA,
            pltpu.SemaphoreType.DMA,
            pltpu.SemaphoreType.DMA,
        ],
    )


def _sc_gather(table, idx_all, coeff_all):
    return _sc_gather_fn()(table, idx_all, coeff_all.reshape(-1))


_PERM = [ (c % 16) * 4 + c // 16 for c in range(64) ]


def kernel(query, value, value_key_padding_mask, value_valid_ratio,
           reference_point, snippet_num, W_so, b_so, W_aw, b_aw, W_v, b_v,
           W_o, b_o):
    Lq, bz, d = query.shape
    Lv = value.shape[0]

    perm = jnp.array(_PERM, dtype=jnp.int32)
    w_cat = jnp.concatenate([W_so[perm], W_aw[perm]], axis=0)
    b_cat = jnp.concatenate([b_so[perm], b_aw[perm]])

    maskf = 1.0 - value_key_padding_mask.T.reshape(Lv * bz, 1).astype(jnp.float32)
    table = _vproj_pair3(value, W_v, b_v, maskf)

    refq = jnp.transpose(reference_point, (1, 0, 2)).reshape(Lq * bz, 2)
    idx_all, coeff_all = _prep(query, w_cat, b_cat, refq)

    attn = _sc_gather(table, idx_all, coeff_all)

    return _outproj3d(attn, W_o, b_o)


def _outproj3d_kernel(x_ref, w_ref, b_ref, o_ref):
    mm = _mm(x_ref[...], w_ref[...]) + b_ref[...]
    o_ref[...] = mm.reshape(o_ref.shape)


def _outproj3d(x, w, b, bm=512):
    M, K = x.shape
    N = w.shape[0]
    grid = (M // bm,)
    return pl.pallas_call(
        _outproj3d_kernel,
        grid=grid,
        in_specs=[
            pl.BlockSpec((bm, K), lambda i: (i, 0)),
            pl.BlockSpec((N, K), lambda i: (0, 0)),
            pl.BlockSpec((1, N), lambda i: (0, 0)),
        ],
        out_specs=pl.BlockSpec((bm // BZ, BZ, N), lambda i: (i, 0, 0)),
        out_shape=jax.ShapeDtypeStruct((M // BZ, BZ, N), jnp.float32),
    )(x, w, b.reshape(1, N))
